# baseline (speedup 1.0000x reference)
import functools

import jax
import jax.numpy as jnp
from jax import lax
from jax.experimental import pallas as pl
from jax.experimental.pallas import tpu as pltpu
from jax.experimental.pallas import tpu_sc as plsc

_NC = 2
_NS = 16
_NW = _NC * _NS
_L = 16


def _sc_partial_max(tp_hbm, out_hbm, buf, vout):
    c = lax.axis_index("c")
    s = lax.axis_index("s")
    wid = s * _NC + c
    ch = buf.shape[0]
    pltpu.sync_copy(tp_hbm.at[pl.ds(wid * ch, ch)], buf)

    def body(i, m):
        return jnp.maximum(m, buf[pl.ds(i * _L, _L)])

    m = lax.fori_loop(1, ch // _L, body, buf[pl.ds(0, _L)])
    vout[...] = m
    pltpu.sync_copy(vout, out_hbm.at[pl.ds(wid * _L, _L)])


def _tc_loss(part_ref, p_ref, g_ref, out_ref):
    thresh = jnp.max(part_ref[...])
    p = p_ref[...]
    g = g_ref[...]
    bce = g * jnp.log(p) + (1.0 - g) * jnp.log(1.0 - p)
    fltr = jnp.logical_and(p > thresh, g == 0.0)
    loss = jnp.where(fltr, bce * 20.0, bce)
    out_ref[...] = (jnp.sum(loss) / p.size).reshape(1, 1)


def kernel(prediction, ground, trailing_pred, trailing_ground):
    n = prediction.shape[0]
    m = trailing_pred.shape[0]
    ch = m // _NW

    sc_max = pl.kernel(
        _sc_partial_max,
        mesh=plsc.VectorSubcoreMesh(core_axis_name="c", subcore_axis_name="s"),
        out_type=jax.ShapeDtypeStruct((_NW * _L,), jnp.float32),
        scratch_types=[
            pltpu.VMEM((ch,), jnp.float32),
            pltpu.VMEM((_L,), jnp.float32),
        ],
    )
    partials = sc_max(trailing_pred)

    p2 = prediction.reshape(n // 128, 128)
    g2 = ground.reshape(n // 128, 128)
    part2 = partials.reshape(_NW * _L // 128, 128)
    out = pl.pallas_call(
        _tc_loss,
        out_shape=jax.ShapeDtypeStruct((1, 1), jnp.float32),
    )(part2, p2, g2)
    return out[0, 0]


# speedup vs baseline: 11.0310x; 11.0310x over previous
import jax
import jax.numpy as jnp
from jax.experimental import pallas as pl


def _loss_kernel(p_ref, g_ref, tp_ref, out_ref):
    thresh = jnp.max(tp_ref[...])
    p = p_ref[...]
    g = g_ref[...]
    bce = jnp.log(g * p + (1.0 - g) * (1.0 - p))
    fltr = jnp.logical_and(p > thresh, g == 0.0)
    loss = jnp.where(fltr, bce * 20.0, bce)
    out_ref[...] = (jnp.sum(loss) / p.size).reshape(1, 1)


def kernel(prediction, ground, trailing_pred, trailing_ground):
    n = prediction.shape[0]
    m = trailing_pred.shape[0]
    p2 = prediction.reshape(n // 128, 128)
    g2 = ground.reshape(n // 128, 128)
    tp2 = trailing_pred.reshape(m // 128, 128)
    out = pl.pallas_call(
        _loss_kernel,
        out_shape=jax.ShapeDtypeStruct((1, 1), jnp.float32),
    )(p2, g2, tp2)
    return out[0, 0]
